# baseline (speedup 1.0000x reference)
import functools

import jax
import jax.numpy as jnp
from jax import lax
from jax.experimental import pallas as pl
from jax.experimental.pallas import tpu as pltpu
from jax.experimental.pallas import tpu_sc as plsc

_B, _C, _DP = 4, 2048, 256
_DR = 64
_TOPK = 8
_TQ = 256

_NBB = _C * _TOPK
_NW = 32
_BPW = _NBB // _NW
_CHUNK = 128
_NCHUNK = _BPW // _CHUNK


def _logits_topk_body(boff_ref, q_ref, k_ref, wq_ref, wk_ref,
                      logits_ref, idx_ref, xk_ref):
    i = pl.program_id(0)

    @pl.when(i == 0)
    def _():
        xk_ref[...] = jnp.dot(
            k_ref[...], wk_ref[...], preferred_element_type=jnp.float32
        )

    x_q = jnp.dot(q_ref[...], wq_ref[...], preferred_element_type=jnp.float32)
    logits = lax.dot_general(
        x_q, xk_ref[...], (((1,), (1,)), ((), ())),
        preferred_element_type=jnp.float32,
    )
    logits_ref[...] = logits

    colf = lax.broadcasted_iota(jnp.int32, (_TQ, _C), 1).astype(jnp.float32)
    t_iota = lax.broadcasted_iota(jnp.int32, (_TQ, _TOPK), 1)
    work = logits
    idx_acc = jnp.zeros((_TQ, _TOPK), jnp.int32)
    for t in range(_TOPK):
        m = jnp.max(work, axis=1, keepdims=True)
        amaxf = jnp.min(jnp.where(work == m, colf, float(_C)), axis=1,
                        keepdims=True)
        idx_acc = jnp.where(t_iota == t, amaxf.astype(jnp.int32), idx_acc)
        work = jnp.where(colf == amaxf, -jnp.inf, work)
    idx_ref[...] = idx_acc + boff_ref[0]


def _logits_topk_b(boff, qb, kb, Wq, Wk, interpret=False):
    return pl.pallas_call(
        _logits_topk_body,
        grid=(_C // _TQ,),
        in_specs=[
            pl.BlockSpec(memory_space=pltpu.SMEM),
            pl.BlockSpec((_TQ, _DP), lambda i: (i, 0)),
            pl.BlockSpec((_C, _DP), lambda i: (0, 0)),
            pl.BlockSpec((_DP, _DR), lambda i: (0, 0)),
            pl.BlockSpec((_DP, _DR), lambda i: (0, 0)),
        ],
        out_specs=[
            pl.BlockSpec((_TQ, _C), lambda i: (i, 0)),
            pl.BlockSpec((_TQ, _TOPK), lambda i: (i, 0)),
        ],
        out_shape=[
            jax.ShapeDtypeStruct((_C, _C), jnp.float32),
            jax.ShapeDtypeStruct((_C, _TOPK), jnp.int32),
        ],
        scratch_shapes=[pltpu.VMEM((_C, _DR), jnp.float32)],
        interpret=interpret,
    )(boff, qb, kb, Wq, Wk)


def _gather_body(v_hbm, idx_hbm, out_hbm, idx_v, buf0, buf1, sem0, sem1):
    wid = lax.axis_index("s") * 2 + lax.axis_index("c")
    base = wid * _BPW
    pltpu.sync_copy(idx_hbm.at[pl.ds(base, _BPW)], idx_v)

    bufs = (buf0, buf1)
    sems = (sem0, sem1)

    def start(g):
        return pltpu.async_copy(
            v_hbm.at[idx_v.at[pl.ds(g * _CHUNK, _CHUNK)]],
            bufs[g % 2],
            sems[g % 2],
        )

    handles = [None, None]
    for g in range(_NCHUNK):
        slot = g % 2
        if handles[slot] is not None:
            handles[slot].wait()
            pltpu.sync_copy(
                bufs[slot], out_hbm.at[pl.ds(base + (g - 2) * _CHUNK, _CHUNK)]
            )
        handles[slot] = start(g)
    for g in range(max(_NCHUNK - 2, 0), _NCHUNK):
        slot = g % 2
        handles[slot].wait()
        pltpu.sync_copy(
            bufs[slot], out_hbm.at[pl.ds(base + g * _CHUNK, _CHUNK)]
        )


@functools.cache
def _gather_rows():
    return functools.partial(
        pl.kernel,
        out_type=jax.ShapeDtypeStruct((_NBB, _DP), jnp.float32),
        mesh=plsc.VectorSubcoreMesh(core_axis_name="c", subcore_axis_name="s"),
        scratch_types=[
            pltpu.VMEM((_BPW,), jnp.int32),
            pltpu.VMEM((_CHUNK, _DP), jnp.float32),
            pltpu.VMEM((_CHUNK, _DP), jnp.float32),
            pltpu.SemaphoreType.DMA,
            pltpu.SemaphoreType.DMA,
        ],
    )(_gather_body)


def kernel(q, k, v, Wq, Wk):
    v_flat = v.reshape(_B * _C, _DP)
    gather = _gather_rows()
    logits_parts = []
    rec_parts = []
    for b in range(_B):
        boff = jnp.full((1,), b * _C, dtype=jnp.int32)
        logits_b, idx_b = _logits_topk_b(boff, q[b], k[b], Wq, Wk)
        logits_parts.append(logits_b)
        rec_parts.append(gather(v_flat, idx_b.reshape(_NBB)))
    logits = jnp.stack(logits_parts, axis=0)
    rec_x = jnp.stack(rec_parts, axis=0).reshape(_B, _C, _TOPK, _DP)
    return (q, rec_x, logits)


# speedup vs baseline: 1.3228x; 1.3228x over previous
import functools

import jax
import jax.numpy as jnp
from jax import lax
from jax.experimental import pallas as pl
from jax.experimental.pallas import tpu as pltpu
from jax.experimental.pallas import tpu_sc as plsc

_B, _C, _DP = 4, 2048, 256
_DR = 64
_TOPK = 8
_TQ = 256

_NB = _B * _C * _TOPK
_NW = 32
_BPW = _NB // _NW
_CHUNK = 128
_NCHUNK = _BPW // _CHUNK


def _logits_topk_body(q_ref, k_ref, wq_ref, wk_ref, logits_ref, idx_ref, xk_ref):
    b = pl.program_id(0)
    i = pl.program_id(1)

    @pl.when(i == 0)
    def _():
        xk_ref[...] = jnp.dot(
            k_ref[0], wk_ref[...], preferred_element_type=jnp.float32
        )

    x_q = jnp.dot(q_ref[0], wq_ref[...], preferred_element_type=jnp.float32)
    logits = lax.dot_general(
        x_q, xk_ref[...], (((1,), (1,)), ((), ())),
        preferred_element_type=jnp.float32,
    )
    logits_ref[0] = logits

    colf = lax.broadcasted_iota(jnp.int32, (_TQ, _C), 1).astype(jnp.float32)
    t_iota = lax.broadcasted_iota(jnp.int32, (_TQ, _TOPK), 1)
    work = logits
    idx_acc = jnp.zeros((_TQ, _TOPK), jnp.int32)
    for t in range(_TOPK):
        m = jnp.max(work, axis=1, keepdims=True)
        amaxf = jnp.min(jnp.where(work == m, colf, float(_C)), axis=1,
                        keepdims=True)
        idx_acc = jnp.where(t_iota == t, amaxf.astype(jnp.int32), idx_acc)
        work = jnp.where(colf == amaxf, -jnp.inf, work)
    idx_ref[0] = idx_acc + b * _C


def _logits_topk(q, k, Wq, Wk, interpret=False):
    return pl.pallas_call(
        _logits_topk_body,
        grid=(_B, _C // _TQ),
        in_specs=[
            pl.BlockSpec((1, _TQ, _DP), lambda b, i: (b, i, 0)),
            pl.BlockSpec((1, _C, _DP), lambda b, i: (b, 0, 0)),
            pl.BlockSpec((_DP, _DR), lambda b, i: (0, 0)),
            pl.BlockSpec((_DP, _DR), lambda b, i: (0, 0)),
        ],
        out_specs=[
            pl.BlockSpec((1, _TQ, _C), lambda b, i: (b, i, 0)),
            pl.BlockSpec((1, _TQ, _TOPK), lambda b, i: (b, i, 0)),
        ],
        out_shape=[
            jax.ShapeDtypeStruct((_B, _C, _C), jnp.float32),
            jax.ShapeDtypeStruct((_B, _C, _TOPK), jnp.int32),
        ],
        scratch_shapes=[pltpu.VMEM((_C, _DR), jnp.float32)],
        interpret=interpret,
    )(q, k, Wq, Wk)


def _gather_body(v_hbm, idx_hbm, out_hbm, idx_v, buf0, buf1, sem0, sem1):
    wid = lax.axis_index("s") * 2 + lax.axis_index("c")
    base = wid * _BPW
    pltpu.sync_copy(idx_hbm.at[pl.ds(base, _BPW)], idx_v)

    bufs = (buf0, buf1)
    sems = (sem0, sem1)

    def start(g):
        return pltpu.async_copy(
            v_hbm.at[idx_v.at[pl.ds(g * _CHUNK, _CHUNK)]],
            bufs[g % 2],
            sems[g % 2],
        )

    handles = [None, None]
    for g in range(_NCHUNK):
        slot = g % 2
        if handles[slot] is not None:
            handles[slot].wait()
            pltpu.sync_copy(
                bufs[slot], out_hbm.at[pl.ds(base + (g - 2) * _CHUNK, _CHUNK)]
            )
        handles[slot] = start(g)
    for g in range(max(_NCHUNK - 2, 0), _NCHUNK):
        slot = g % 2
        handles[slot].wait()
        pltpu.sync_copy(
            bufs[slot], out_hbm.at[pl.ds(base + g * _CHUNK, _CHUNK)]
        )


@functools.cache
def _gather_rows():
    return functools.partial(
        pl.kernel,
        out_type=jax.ShapeDtypeStruct((_NB, _DP), jnp.float32),
        mesh=plsc.VectorSubcoreMesh(core_axis_name="c", subcore_axis_name="s"),
        scratch_types=[
            pltpu.VMEM((_BPW,), jnp.int32),
            pltpu.VMEM((_CHUNK, _DP), jnp.float32),
            pltpu.VMEM((_CHUNK, _DP), jnp.float32),
            pltpu.SemaphoreType.DMA,
            pltpu.SemaphoreType.DMA,
        ],
    )(_gather_body)


def kernel(q, k, v, Wq, Wk):
    logits, idx_global = _logits_topk(q, k, Wq, Wk)
    flat_idx = idx_global.reshape(_NB)
    v_flat = v.reshape(_B * _C, _DP)
    rec_flat = _gather_rows()(v_flat, flat_idx)
    rec_x = rec_flat.reshape(_B, _C, _TOPK, _DP)
    return (q, rec_x, logits)


# speedup vs baseline: 1.3562x; 1.0253x over previous
import functools

import jax
import jax.numpy as jnp
from jax import lax
from jax.experimental import pallas as pl
from jax.experimental.pallas import tpu as pltpu
from jax.experimental.pallas import tpu_sc as plsc

_B, _C, _DP = 4, 2048, 256
_DR = 64
_TOPK = 8
_TQ = 512

_NB = _B * _C * _TOPK
_NW = 32
_BPW = _NB // _NW
_CHUNK = 128
_NCHUNK = _BPW // _CHUNK


def _logits_topk_body(q_ref, k_ref, wq_ref, wk_ref, logits_ref, idx_ref, xk_ref):
    b = pl.program_id(0)
    i = pl.program_id(1)

    @pl.when(i == 0)
    def _():
        xk_ref[...] = jnp.dot(
            k_ref[0], wk_ref[...], preferred_element_type=jnp.float32
        )

    x_q = jnp.dot(q_ref[0], wq_ref[...], preferred_element_type=jnp.float32)
    logits = lax.dot_general(
        x_q, xk_ref[...], (((1,), (1,)), ((), ())),
        preferred_element_type=jnp.float32,
    )
    logits_ref[0] = logits

    colf = lax.broadcasted_iota(jnp.int32, (_TQ, _C), 1).astype(jnp.float32)
    t_iota = lax.broadcasted_iota(jnp.int32, (_TQ, _TOPK), 1)
    work = logits
    idx_acc = jnp.zeros((_TQ, _TOPK), jnp.int32)
    for t in range(_TOPK):
        m = jnp.max(work, axis=1, keepdims=True)
        amaxf = jnp.min(jnp.where(work == m, colf, float(_C)), axis=1,
                        keepdims=True)
        idx_acc = jnp.where(t_iota == t, amaxf.astype(jnp.int32), idx_acc)
        work = jnp.where(colf == amaxf, -jnp.inf, work)
    idx_ref[0] = idx_acc + b * _C


def _logits_topk(q, k, Wq, Wk, interpret=False):
    return pl.pallas_call(
        _logits_topk_body,
        grid=(_B, _C // _TQ),
        in_specs=[
            pl.BlockSpec((1, _TQ, _DP), lambda b, i: (b, i, 0)),
            pl.BlockSpec((1, _C, _DP), lambda b, i: (b, 0, 0)),
            pl.BlockSpec((_DP, _DR), lambda b, i: (0, 0)),
            pl.BlockSpec((_DP, _DR), lambda b, i: (0, 0)),
        ],
        out_specs=[
            pl.BlockSpec((1, _TQ, _C), lambda b, i: (b, i, 0)),
            pl.BlockSpec((1, _TQ, _TOPK), lambda b, i: (b, i, 0)),
        ],
        out_shape=[
            jax.ShapeDtypeStruct((_B, _C, _C), jnp.float32),
            jax.ShapeDtypeStruct((_B, _C, _TOPK), jnp.int32),
        ],
        scratch_shapes=[pltpu.VMEM((_C, _DR), jnp.float32)],
        interpret=interpret,
    )(q, k, Wq, Wk)


_NBUF = 3


def _gather_body(v_hbm, idx_hbm, out_hbm, idx_v, buf0, buf1, buf2,
                 gsem0, gsem1, gsem2, wsem0, wsem1, wsem2):
    wid = lax.axis_index("s") * 2 + lax.axis_index("c")
    base = wid * _BPW
    pltpu.sync_copy(idx_hbm.at[pl.ds(base, _BPW)], idx_v)

    bufs = (buf0, buf1, buf2)
    gsems = (gsem0, gsem1, gsem2)
    wsems = (wsem0, wsem1, wsem2)

    def start_gather(g):
        s = g % _NBUF
        return pltpu.async_copy(
            v_hbm.at[idx_v.at[pl.ds(g * _CHUNK, _CHUNK)]], bufs[s], gsems[s]
        )

    def start_write(g):
        s = g % _NBUF
        return pltpu.async_copy(
            bufs[s], out_hbm.at[pl.ds(base + g * _CHUNK, _CHUNK)], wsems[s]
        )

    gh = [None] * _NBUF
    wh = [None] * _NBUF
    for g in range(_NCHUNK):
        s = g % _NBUF
        if wh[s] is not None:
            wh[s].wait()
        gh[s] = start_gather(g)
        if g >= 1:
            sp = (g - 1) % _NBUF
            gh[sp].wait()
            wh[sp] = start_write(g - 1)
    last = (_NCHUNK - 1) % _NBUF
    gh[last].wait()
    wh[last] = start_write(_NCHUNK - 1)
    for s in range(_NBUF):
        if wh[s] is not None:
            wh[s].wait()


@functools.cache
def _gather_rows():
    return functools.partial(
        pl.kernel,
        out_type=jax.ShapeDtypeStruct((_NB, _DP), jnp.float32),
        mesh=plsc.VectorSubcoreMesh(core_axis_name="c", subcore_axis_name="s"),
        scratch_types=(
            [pltpu.VMEM((_BPW,), jnp.int32)]
            + [pltpu.VMEM((_CHUNK, _DP), jnp.float32)] * _NBUF
            + [pltpu.SemaphoreType.DMA] * (2 * _NBUF)
        ),
    )(_gather_body)


def kernel(q, k, v, Wq, Wk):
    logits, idx_global = _logits_topk(q, k, Wq, Wk)
    flat_idx = idx_global.reshape(_NB)
    v_flat = v.reshape(_B * _C, _DP)
    rec_flat = _gather_rows()(v_flat, flat_idx)
    rec_x = rec_flat.reshape(_B, _C, _TOPK, _DP)
    return (q, rec_x, logits)
